# baseline (speedup 1.0000x reference)
import jax
import jax.numpy as jnp
from jax import lax
from jax.experimental import pallas as pl
from jax.experimental.pallas import tpu as pltpu
from jax.experimental.pallas import tpu_sc as plsc

B = 16384
L = 20
D = 128
NC = 2
NS = 16
NW = NC * NS
RPW = B // NW
UI_CHUNK = 128
HC = 16
NHC = RPW // HC
TROWS = 500


def _sc_body(uidx_hbm, iidx_hbm, hidx_hbm, utab, ctab, htab, jout,
             uidx_v, iidx_v, hidx_v, tbl, rbuf, habuf,
             tgs, ugs0, ugs1, uws0, uws1, hos0, hos1):
    c = lax.axis_index("c")
    s = lax.axis_index("s")
    wid = c * NS + s
    base = wid * RPW

    with jax.named_scope("idx_stage"):
        cp_t = pltpu.async_copy(htab.at[pl.ds(0, TROWS)], tbl, tgs)
        cp_u = pltpu.async_copy(uidx_hbm.at[pl.ds(base, RPW)], uidx_v, ugs0)
        cp_i = pltpu.async_copy(iidx_hbm.at[pl.ds(base, RPW)], iidx_v, ugs1)
        cp_h = pltpu.async_copy(hidx_hbm.at[pl.ds(base * L, RPW * L)],
                                hidx_v, uws0)
        cp_u.wait()
        cp_i.wait()
        cp_h.wait()

    def drain(src, dst, sem):
        pltpu.make_async_copy(src, dst, sem).wait()

    plan = ([(utab, uidx_v, 0, k) for k in range(RPW // UI_CHUNK)] +
            [(ctab, iidx_v, D, k) for k in range(RPW // UI_CHUNK)])
    gse = [ugs0, ugs1]
    wse = [uws0, uws1]
    NP = len(plan)

    def fire_ui(t):
        tab, idxv, _, k = plan[t]
        pltpu.async_copy(
            tab.at[idxv.at[pl.ds(k * UI_CHUNK, UI_CHUNK)]], rbuf.at[t % 2],
            gse[t % 2])

    def fire_ui_out(t):
        _, _, col, k = plan[t]
        pltpu.async_copy(
            rbuf.at[t % 2],
            jout.at[pl.ds(base + k * UI_CHUNK, UI_CHUNK), pl.ds(col, D)],
            wse[t % 2])

    def ui_event(t):
        if t < NP:
            if t >= 2:
                drain(utab.at[pl.ds(0, UI_CHUNK)], rbuf.at[t % 2], wse[t % 2])
            fire_ui(t)
        if 1 <= t <= NP:
            drain(utab.at[pl.ds(0, UI_CHUNK)], rbuf.at[(t - 1) % 2],
                  gse[(t - 1) % 2])
            fire_ui_out(t - 1)

    def rows16(b0, hb):
        def rowpair(j, _):
            bb = (b0 + 2 * j) * L
            v0 = hidx_v[pl.ds(bb, 16)]
            v1 = hidx_v[pl.ds(bb + 16, 16)]
            v2 = hidx_v[pl.ds(bb + 24, 16)]
            h_a = [v0[l] for l in range(16)] + [v1[l] for l in range(4)]
            h_b = [v1[l] for l in range(4, 16)] + [v2[l] for l in range(8, 16)]
            for r, hsc in ((2 * j, h_a), (2 * j + 1, h_b)):
                rr = [h >> 1 for h in hsc]
                cc = [(h & 1) * 64 for h in hsc]
                for g in range(D // 32):
                    w = tbl[rr[0], pl.ds(cc[0] + g * 16, 16)]
                    acc_lo = lax.bitcast_convert_type(w << 16, jnp.float32)
                    acc_hi = lax.bitcast_convert_type(w, jnp.float32)
                    for l in range(1, L):
                        w = tbl[rr[l], pl.ds(cc[l] + g * 16, 16)]
                        acc_lo = acc_lo + lax.bitcast_convert_type(w << 16, jnp.float32)
                        acc_hi = acc_hi + lax.bitcast_convert_type(w, jnp.float32)
                    hb[r, pl.ds(g * 32, 16)] = acc_lo
                    hb[r, pl.ds(g * 32 + 16, 16)] = acc_hi
            return 0
        lax.fori_loop(0, HC // 2, rowpair, 0)

    hb0 = habuf.at[0]
    hb1 = habuf.at[1]

    ui_event(0)
    ui_event(1)
    cp_t.wait()

    def pair(i, _):
        a = 2 * i
        b = a + 1
        for t in range(2, NP + 1):
            @pl.when(i == t - 2)
            def _(t=t):
                ui_event(t)

        @pl.when(i > 0)
        def _():
            drain(utab.at[pl.ds(0, HC)], hb0, hos0)
        rows16(a * HC, hb0)
        pltpu.async_copy(
            hb0, jout.at[pl.ds(base + a * HC, HC), pl.ds(2 * D, D)], hos0)

        @pl.when(i > 0)
        def _():
            drain(utab.at[pl.ds(0, HC)], hb1, hos1)
        rows16(b * HC, hb1)
        pltpu.async_copy(
            hb1, jout.at[pl.ds(base + b * HC, HC), pl.ds(2 * D, D)], hos1)
        return 0

    with jax.named_scope("hist_pool"):
        lax.fori_loop(0, NHC // 2, pair, 0)
        drain(utab.at[pl.ds(0, HC)], hb0, hos0)
        drain(utab.at[pl.ds(0, HC)], hb1, hos1)
        drain(utab.at[pl.ds(0, UI_CHUNK)], rbuf.at[0], wse[0])
        drain(utab.at[pl.ds(0, UI_CHUNK)], rbuf.at[1], wse[1])


@jax.jit
def _sc_gather(uidx, iidx, hidx, utab, ctab, htab):
    mesh = plsc.VectorSubcoreMesh(core_axis_name="c", subcore_axis_name="s")
    f = pl.kernel(
        _sc_body,
        out_type=jax.ShapeDtypeStruct((B, 3 * D), jnp.float32),
        mesh=mesh,
        scratch_types=[
            pltpu.VMEM((RPW,), jnp.int32),
            pltpu.VMEM((RPW,), jnp.int32),
            pltpu.VMEM((L * RPW,), jnp.int32),
            pltpu.VMEM((TROWS, D), jnp.int32),
            pltpu.VMEM((2, UI_CHUNK, D), jnp.float32),
            pltpu.VMEM((2, HC, D), jnp.float32),
        ] + [pltpu.SemaphoreType.DMA] * 7,
    )
    return f(uidx, iidx, hidx, utab, ctab, htab)


BLK = 2048


def _head_body(x_ref, g_ref, be_ref,
               w1_ref, b1_ref, a1_ref, w2_ref, b2_ref, a2_ref,
               w3_ref, b3_ref, o_ref, acc_s, acc_q):
    ph = pl.program_id(0)

    @pl.when(ph == 0)
    def _():
        @pl.when(pl.program_id(1) == 0)
        def _():
            acc_s[...] = jnp.zeros_like(acc_s)
            acc_q[...] = jnp.zeros_like(acc_q)
        x = x_ref[...]
        acc_s[...] += jnp.sum(x, axis=0, keepdims=True)
        acc_q[...] += jnp.sum(x * x, axis=0, keepdims=True)

    @pl.when(ph == 1)
    def _():
        inv_b = 1.0 / B
        mean = acc_s[...] * inv_b
        var = acc_q[...] * inv_b - mean * mean
        inv = lax.rsqrt(var + 1e-5)
        scale = g_ref[...] * inv
        shift = be_ref[...] - mean * scale
        xh = (x_ref[...] * scale + shift).astype(jnp.bfloat16)
        h1 = b1_ref[...] + jnp.dot(xh, w1_ref[...],
                                   preferred_element_type=jnp.float32)
        a1 = a1_ref[0, 0]
        h1 = (jnp.maximum(h1, 0.0) + a1 * jnp.minimum(h1, 0.0)).astype(jnp.bfloat16)
        a2 = a2_ref[0, 0]
        h2 = jnp.dot(h1, w2_ref[...], preferred_element_type=jnp.float32) + b2_ref[...]
        h2 = (jnp.maximum(h2, 0.0) + a2 * jnp.minimum(h2, 0.0)).astype(jnp.bfloat16)
        lg = jnp.dot(h2, w3_ref[...], preferred_element_type=jnp.float32) + b3_ref[...]
        m = jnp.max(lg, axis=1, keepdims=True)
        e = jnp.exp(lg - m)
        p = e / jnp.sum(e, axis=1, keepdims=True)
        o_ref[...] = p.T


def _tc_head(j_rows, bn_gamma, bn_beta, W1, b1, a1, W2, b2, a2, W3, b3):
    nblk = B // BLK
    blk = lambda: pl.BlockSpec((BLK, 3 * D), lambda p, i: (i, 0))
    full = lambda r, c: pl.BlockSpec((r, c), lambda p, i: (0, 0))
    out = pl.pallas_call(
        _head_body,
        grid=(2, nblk),
        in_specs=[blk(),
                  full(1, 384), full(1, 384),
                  full(384, 200), full(1, 200), full(1, 1),
                  full(200, 80), full(1, 80), full(1, 1),
                  full(80, 2), full(1, 2)],
        out_specs=pl.BlockSpec((2, BLK), lambda p, i: (0, i)),
        out_shape=jax.ShapeDtypeStruct((2, B), jnp.float32),
        scratch_shapes=[pltpu.VMEM((1, 384), jnp.float32)] * 2,
    )(j_rows,
      bn_gamma.reshape(1, 384), bn_beta.reshape(1, 384),
      W1.astype(jnp.bfloat16), b1.reshape(1, 200), a1.reshape(1, 1),
      W2.astype(jnp.bfloat16), b2.reshape(1, 80), a2.reshape(1, 1),
      W3.astype(jnp.bfloat16), b3.reshape(1, 2))
    return out.T


def kernel(user, item, history, length, user_table, item_table, cate_table,
           cate_list, bn_gamma, bn_beta, W1, b1, a1, W2, b2, a2, W3, b3):
    ctab = jnp.concatenate(
        [item_table, jnp.take(cate_table, cate_list, axis=0)], axis=1)
    g = jnp.arange(D)
    p = g % 32
    q = (p % 2) * 16 + p // 2
    hperm = (g // 32) * 32 + q
    htab = lax.bitcast_convert_type(
        ctab[:, hperm].astype(jnp.bfloat16).reshape(-1, D // 2, 2),
        jnp.int32).reshape(TROWS, D)
    j_rows = _sc_gather(
        user.astype(jnp.int32), item.astype(jnp.int32),
        history.reshape(-1).astype(jnp.int32), user_table, ctab, htab)
    return _tc_head(j_rows, bn_gamma, bn_beta,
                    W1, b1, a1, W2, b2, a2, W3, b3)


# speedup vs baseline: 1.0160x; 1.0160x over previous
import jax
import jax.numpy as jnp
from jax import lax
from jax.experimental import pallas as pl
from jax.experimental.pallas import tpu as pltpu
from jax.experimental.pallas import tpu_sc as plsc

B = 16384
L = 20
D = 128
NC = 2
NS = 16
NW = NC * NS
RPW = B // NW
UI_CHUNK = 128
HC = 16
NHC = RPW // HC
TROWS = 500


def _sc_body(uidx_hbm, iidx_hbm, hidx_hbm, utab, ctab, htab, jout,
             uidx_v, iidx_v, hidx_v, tbl, rbuf, habuf,
             tgs, ugs0, ugs1, uws0, uws1, hos0, hos1):
    c = lax.axis_index("c")
    s = lax.axis_index("s")
    wid = c * NS + s
    base = wid * RPW

    with jax.named_scope("idx_stage"):
        cp_t = pltpu.async_copy(htab.at[pl.ds(0, TROWS)], tbl, tgs)
        cp_u = pltpu.async_copy(uidx_hbm.at[pl.ds(base, RPW)], uidx_v, ugs0)
        cp_i = pltpu.async_copy(iidx_hbm.at[pl.ds(base, RPW)], iidx_v, ugs1)
        cp_h = pltpu.async_copy(hidx_hbm.at[pl.ds(base * L, RPW * L)],
                                hidx_v, uws0)
        cp_u.wait()
        cp_i.wait()
        cp_h.wait()

    def drain(src, dst, sem):
        pltpu.make_async_copy(src, dst, sem).wait()

    plan = ([(utab, uidx_v, 0, k) for k in range(RPW // UI_CHUNK)] +
            [(ctab, iidx_v, D, k) for k in range(RPW // UI_CHUNK)])
    gse = [ugs0, ugs1]
    wse = [uws0, uws1]
    NP = len(plan)

    def fire_ui(t):
        tab, idxv, _, k = plan[t]
        pltpu.async_copy(
            tab.at[idxv.at[pl.ds(k * UI_CHUNK, UI_CHUNK)]], rbuf.at[t % 2],
            gse[t % 2])

    def fire_ui_out(t):
        _, _, col, k = plan[t]
        pltpu.async_copy(
            rbuf.at[t % 2],
            jout.at[pl.ds(base + k * UI_CHUNK, UI_CHUNK), pl.ds(col, D)],
            wse[t % 2])

    def ui_event(t):
        if t < NP:
            if t >= 2:
                drain(utab.at[pl.ds(0, UI_CHUNK)], rbuf.at[t % 2], wse[t % 2])
            fire_ui(t)
        if 1 <= t <= NP:
            drain(utab.at[pl.ds(0, UI_CHUNK)], rbuf.at[(t - 1) % 2],
                  gse[(t - 1) % 2])
            fire_ui_out(t - 1)

    def rows16(b0, hb):
        def rowpair(j, _):
            bb = (b0 + 2 * j) * L
            v0 = hidx_v[pl.ds(bb, 16)]
            v1 = hidx_v[pl.ds(bb + 16, 16)]
            v2 = hidx_v[pl.ds(bb + 24, 16)]
            h_a = [v0[l] for l in range(16)] + [v1[l] for l in range(4)]
            h_b = [v1[l] for l in range(4, 16)] + [v2[l] for l in range(8, 16)]
            for r, hsc in ((2 * j, h_a), (2 * j + 1, h_b)):
                rr = [h >> 1 for h in hsc]
                cc = [(h & 1) * 64 for h in hsc]
                for g in range(D // 32):
                    w = tbl[rr[0], pl.ds(cc[0] + g * 16, 16)]
                    acc_lo = lax.bitcast_convert_type(w << 16, jnp.float32)
                    acc_hi = lax.bitcast_convert_type(w, jnp.float32)
                    for l in range(1, L):
                        w = tbl[rr[l], pl.ds(cc[l] + g * 16, 16)]
                        acc_lo = acc_lo + lax.bitcast_convert_type(w << 16, jnp.float32)
                        acc_hi = acc_hi + lax.bitcast_convert_type(w, jnp.float32)
                    hb[r, pl.ds(g * 32, 16)] = acc_lo
                    hb[r, pl.ds(g * 32 + 16, 16)] = acc_hi
            return 0
        lax.fori_loop(0, HC // 2, rowpair, 0)

    hb0 = habuf.at[0]
    hb1 = habuf.at[1]

    ui_event(0)
    ui_event(1)
    cp_t.wait()

    def pair(i, _):
        a = 2 * i
        b = a + 1
        for t in range(2, NP + 1):
            @pl.when(i == t - 2)
            def _(t=t):
                ui_event(t)

        @pl.when(i > 0)
        def _():
            drain(utab.at[pl.ds(0, HC)], hb0, hos0)
        rows16(a * HC, hb0)
        pltpu.async_copy(
            hb0, jout.at[pl.ds(base + a * HC, HC), pl.ds(2 * D, D)], hos0)

        @pl.when(i > 0)
        def _():
            drain(utab.at[pl.ds(0, HC)], hb1, hos1)
        rows16(b * HC, hb1)
        pltpu.async_copy(
            hb1, jout.at[pl.ds(base + b * HC, HC), pl.ds(2 * D, D)], hos1)
        return 0

    with jax.named_scope("hist_pool"):
        lax.fori_loop(0, NHC // 2, pair, 0)
        drain(utab.at[pl.ds(0, HC)], hb0, hos0)
        drain(utab.at[pl.ds(0, HC)], hb1, hos1)
        drain(utab.at[pl.ds(0, UI_CHUNK)], rbuf.at[0], wse[0])
        drain(utab.at[pl.ds(0, UI_CHUNK)], rbuf.at[1], wse[1])


@jax.jit
def _sc_gather(uidx, iidx, hidx, utab, ctab, htab):
    mesh = plsc.VectorSubcoreMesh(core_axis_name="c", subcore_axis_name="s")
    f = pl.kernel(
        _sc_body,
        out_type=jax.ShapeDtypeStruct((B, 3 * D), jnp.float32),
        mesh=mesh,
        scratch_types=[
            pltpu.VMEM((RPW,), jnp.int32),
            pltpu.VMEM((RPW,), jnp.int32),
            pltpu.VMEM((L * RPW,), jnp.int32),
            pltpu.VMEM((TROWS, D), jnp.int32),
            pltpu.VMEM((2, UI_CHUNK, D), jnp.float32),
            pltpu.VMEM((2, HC, D), jnp.float32),
        ] + [pltpu.SemaphoreType.DMA] * 7,
    )
    return f(uidx, iidx, hidx, utab, ctab, htab)


BLK = 4096


def _head_body(x_ref, g_ref, be_ref,
               w1_ref, b1_ref, a1_ref, w2_ref, b2_ref, a2_ref,
               w3_ref, b3_ref, o_ref, acc_s, acc_q):
    ph = pl.program_id(0)

    @pl.when(ph == 0)
    def _():
        @pl.when(pl.program_id(1) == 0)
        def _():
            acc_s[...] = jnp.zeros_like(acc_s)
            acc_q[...] = jnp.zeros_like(acc_q)
        x = x_ref[...]
        acc_s[...] += jnp.sum(x, axis=0, keepdims=True)
        acc_q[...] += jnp.sum(x * x, axis=0, keepdims=True)

    @pl.when(ph == 1)
    def _():
        inv_b = 1.0 / B
        mean = acc_s[...] * inv_b
        var = acc_q[...] * inv_b - mean * mean
        inv = lax.rsqrt(var + 1e-5)
        scale = g_ref[...] * inv
        shift = be_ref[...] - mean * scale
        xh = (x_ref[...] * scale + shift).astype(jnp.bfloat16)
        h1 = b1_ref[...] + jnp.dot(xh, w1_ref[...],
                                   preferred_element_type=jnp.float32)
        a1 = a1_ref[0, 0]
        h1 = (jnp.maximum(h1, 0.0) + a1 * jnp.minimum(h1, 0.0)).astype(jnp.bfloat16)
        a2 = a2_ref[0, 0]
        h2 = jnp.dot(h1, w2_ref[...], preferred_element_type=jnp.float32) + b2_ref[...]
        h2 = (jnp.maximum(h2, 0.0) + a2 * jnp.minimum(h2, 0.0)).astype(jnp.bfloat16)
        lg = jnp.dot(h2, w3_ref[...], preferred_element_type=jnp.float32) + b3_ref[...]
        m = jnp.max(lg, axis=1, keepdims=True)
        e = jnp.exp(lg - m)
        p = e / jnp.sum(e, axis=1, keepdims=True)
        o_ref[...] = p.T


def _tc_head(j_rows, bn_gamma, bn_beta, W1, b1, a1, W2, b2, a2, W3, b3):
    nblk = B // BLK
    blk = lambda: pl.BlockSpec((BLK, 3 * D), lambda p, i: (i, 0))
    full = lambda r, c: pl.BlockSpec((r, c), lambda p, i: (0, 0))
    out = pl.pallas_call(
        _head_body,
        grid=(2, nblk),
        in_specs=[blk(),
                  full(1, 384), full(1, 384),
                  full(384, 200), full(1, 200), full(1, 1),
                  full(200, 80), full(1, 80), full(1, 1),
                  full(80, 2), full(1, 2)],
        out_specs=pl.BlockSpec((2, BLK), lambda p, i: (0, i)),
        out_shape=jax.ShapeDtypeStruct((2, B), jnp.float32),
        scratch_shapes=[pltpu.VMEM((1, 384), jnp.float32)] * 2,
    )(j_rows,
      bn_gamma.reshape(1, 384), bn_beta.reshape(1, 384),
      W1.astype(jnp.bfloat16), b1.reshape(1, 200), a1.reshape(1, 1),
      W2.astype(jnp.bfloat16), b2.reshape(1, 80), a2.reshape(1, 1),
      W3.astype(jnp.bfloat16), b3.reshape(1, 2))
    return out.T


def kernel(user, item, history, length, user_table, item_table, cate_table,
           cate_list, bn_gamma, bn_beta, W1, b1, a1, W2, b2, a2, W3, b3):
    ctab = jnp.concatenate(
        [item_table, jnp.take(cate_table, cate_list, axis=0)], axis=1)
    g = jnp.arange(D)
    p = g % 32
    q = (p % 2) * 16 + p // 2
    hperm = (g // 32) * 32 + q
    htab = lax.bitcast_convert_type(
        ctab[:, hperm].astype(jnp.bfloat16).reshape(-1, D // 2, 2),
        jnp.int32).reshape(TROWS, D)
    j_rows = _sc_gather(
        user.astype(jnp.int32), item.astype(jnp.int32),
        history.reshape(-1).astype(jnp.int32), user_table, ctab, htab)
    return _tc_head(j_rows, bn_gamma, bn_beta,
                    W1, b1, a1, W2, b2, a2, W3, b3)
